# baseline (speedup 1.0000x reference)
import functools

import jax
import jax.numpy as jnp
from jax import lax
from jax.experimental import pallas as pl
from jax.experimental.pallas import tpu as pltpu
from jax.experimental.pallas import tpu_sc as plsc

N = 32768
H = 128
NCH = N // H


KP = 32
RB = 8


def _ranks_kernel(total_ref, x_ref, hcol_ref, hrow_ref, lidx_ref, acc_ref):
    i = pl.program_id(0)

    for r in range(RB):
        T = total_ref[i * RB + r, 0]

        for m in range(H // KP):
            @pl.when(T > m * KP)
            def _():
                hs = hcol_ref[r, pl.ds(m * KP, KP), :]
                hb = jnp.broadcast_to(hs, (KP, H))
                z = jnp.zeros((KP, H), jnp.float32)
                U = 8

                def body(c, accs):
                    out = []
                    for u in range(U):
                        xu = x_ref[r, pl.ds((U * c + u) * H, H)].reshape(1, H)
                        out.append(accs[u] + (xu > hb).astype(jnp.float32))
                    return tuple(out)

                accs = lax.fori_loop(0, NCH // U, body, (z,) * U)
                acc_ref[pl.ds(m * KP, KP), :] = (
                    ((accs[0] + accs[1]) + (accs[2] + accs[3]))
                    + ((accs[4] + accs[5]) + (accs[6] + accs[7])))

        hcol = hcol_ref[r]
        HB = jnp.broadcast_to(hcol, (H, H))

        hrow = hrow_ref[r]
        ks = lax.broadcasted_iota(jnp.int32, (H, H), 0)
        ml = lax.broadcasted_iota(jnp.int32, (H, H), 1)
        ties = ((HB == hrow) & (ml < ks)).astype(jnp.float32)
        total = acc_ref[...] + ties
        ranks = jnp.sum(total, axis=1, keepdims=True).astype(jnp.int32)

        kcol = lax.broadcasted_iota(jnp.int32, (H, 1), 0)
        lidx_ref[r] = jnp.where(kcol < T, ranks, -1)


def _sc_scatter_body(x_flat, lidx_hbm, out_hbm, lidx_v, gidx_v, vals_v, canvas):
    info = plsc.get_sparse_core_info()
    nc = info.num_cores
    wid = lax.axis_index("s") * nc + lax.axis_index("c")

    zeros16 = jnp.zeros((16,), jnp.float32)

    def zero_body(z, carry):
        base = z * 128
        for u in range(8):
            canvas[pl.ds(base + u * 16, 16)] = zeros16
        return carry

    lax.fori_loop(0, N // 128, zero_body, 0)

    pltpu.sync_copy(lidx_hbm.at[wid], lidx_v)

    for r in range(2):
        row = wid * 2 + r
        for s in range(H // 16):
            sl = pl.ds(s * 16, 16)
            gidx_v[sl] = jnp.maximum(lidx_v[r, sl], 0) + row * N
        pltpu.sync_copy(x_flat.at[gidx_v], vals_v)
        for s in range(H // 16):
            sl = pl.ds(s * 16, 16)
            raw = lidx_v[r, sl]
            plsc.store_scatter(canvas, [jnp.maximum(raw, 0)], vals_v[sl],
                               mask=raw >= 0)
        pltpu.sync_copy(canvas, out_hbm.at[row])
        if r == 0:
            for s in range(H // 16):
                sl = pl.ds(s * 16, 16)
                raw = lidx_v[r, sl]
                plsc.store_scatter(canvas, [jnp.maximum(raw, 0)], zeros16,
                                   mask=raw >= 0)


@jax.jit
def kernel(in_proba, in_total):
    R = in_proba.shape[0]
    hcol = in_proba[:, :H].reshape(R, H, 1)
    hrow = in_proba[:, :H].reshape(R, 1, H)

    lidx = pl.pallas_call(
        _ranks_kernel,
        grid=(R // RB,),
        in_specs=[
            pl.BlockSpec((R, 1), lambda i: (0, 0), memory_space=pltpu.SMEM),
            pl.BlockSpec((RB, N), lambda i: (i, 0)),
            pl.BlockSpec((RB, H, 1), lambda i: (i, 0, 0)),
            pl.BlockSpec((RB, 1, H), lambda i: (i, 0, 0)),
        ],
        out_specs=pl.BlockSpec((RB, H, 1), lambda i: (i, 0, 0)),
        out_shape=jax.ShapeDtypeStruct((R, H, 1), jnp.int32),
        scratch_shapes=[pltpu.VMEM((H, H), jnp.float32)],
        compiler_params=pltpu.CompilerParams(
            dimension_semantics=("parallel",)),
    )(in_total, in_proba, hcol, hrow)

    sc_call = pl.kernel(
        _sc_scatter_body,
        out_type=jax.ShapeDtypeStruct((R, N), jnp.float32),
        mesh=plsc.VectorSubcoreMesh(core_axis_name="c", subcore_axis_name="s"),
        compiler_params=pltpu.CompilerParams(needs_layout_passes=False),
        scratch_types=[
            pltpu.VMEM((2, H), jnp.int32),
            pltpu.VMEM((H,), jnp.int32),
            pltpu.VMEM((H,), jnp.float32),
            pltpu.VMEM((N,), jnp.float32),
        ],
    )
    return sc_call(in_proba.reshape(R * N), lidx.reshape(R // 2, 2, H))


# speedup vs baseline: 1.0105x; 1.0105x over previous
import functools

import jax
import jax.numpy as jnp
from jax import lax
from jax.experimental import pallas as pl
from jax.experimental.pallas import tpu as pltpu
from jax.experimental.pallas import tpu_sc as plsc

N = 32768
H = 128
NCH = N // H


KP = 32
RB = 8


def _ranks_kernel(total_ref, x_ref, hcol_ref, hrow_ref, lidx_ref, acc_ref):
    i = pl.program_id(0)

    for r in range(RB):
        T = total_ref[i * RB + r, 0]

        for m in range(H // KP):
            @pl.when(T > m * KP)
            def _():
                hs = hcol_ref[r, pl.ds(m * KP, KP), :]
                hb = jnp.broadcast_to(hs, (KP, H))
                z = jnp.zeros((KP, H), jnp.float32)
                U = 4

                def body(c, accs):
                    out = []
                    for u in range(U):
                        xu = x_ref[r, pl.ds((U * c + u) * H, H)].reshape(1, H)
                        out.append(accs[u] + (xu > hb).astype(jnp.float32))
                    return tuple(out)

                accs = lax.fori_loop(0, NCH // U, body, (z,) * U)
                acc_ref[pl.ds(m * KP, KP), :] = (
                    (accs[0] + accs[1]) + (accs[2] + accs[3]))

        hcol = hcol_ref[r]
        HB = jnp.broadcast_to(hcol, (H, H))

        hrow = hrow_ref[r]
        ks = lax.broadcasted_iota(jnp.int32, (H, H), 0)
        ml = lax.broadcasted_iota(jnp.int32, (H, H), 1)
        ties = ((HB == hrow) & (ml < ks)).astype(jnp.float32)
        total = acc_ref[...] + ties
        ranks = jnp.sum(total, axis=1, keepdims=True).astype(jnp.int32)

        kcol = lax.broadcasted_iota(jnp.int32, (H, 1), 0)
        lidx_ref[r] = jnp.where(kcol < T, ranks, -1)


def _sc_scatter_body(x_flat, lidx_hbm, out_hbm, lidx_v, gidx_v, vals_v, canvas):
    info = plsc.get_sparse_core_info()
    nc = info.num_cores
    wid = lax.axis_index("s") * nc + lax.axis_index("c")

    zeros16 = jnp.zeros((16,), jnp.float32)

    def zero_body(z, carry):
        base = z * 128
        for u in range(8):
            canvas[pl.ds(base + u * 16, 16)] = zeros16
        return carry

    lax.fori_loop(0, N // 128, zero_body, 0)

    pltpu.sync_copy(lidx_hbm.at[wid], lidx_v)

    for r in range(2):
        row = wid * 2 + r
        for s in range(H // 16):
            sl = pl.ds(s * 16, 16)
            gidx_v[sl] = jnp.maximum(lidx_v[r, sl], 0) + row * N
        pltpu.sync_copy(x_flat.at[gidx_v], vals_v)
        for s in range(H // 16):
            sl = pl.ds(s * 16, 16)
            raw = lidx_v[r, sl]
            plsc.store_scatter(canvas, [jnp.maximum(raw, 0)], vals_v[sl],
                               mask=raw >= 0)
        pltpu.sync_copy(canvas, out_hbm.at[row])
        if r == 0:
            for s in range(H // 16):
                sl = pl.ds(s * 16, 16)
                raw = lidx_v[r, sl]
                plsc.store_scatter(canvas, [jnp.maximum(raw, 0)], zeros16,
                                   mask=raw >= 0)


@jax.jit
def kernel(in_proba, in_total):
    R = in_proba.shape[0]
    hcol = in_proba[:, :H].reshape(R, H, 1)
    hrow = in_proba[:, :H].reshape(R, 1, H)

    lidx = pl.pallas_call(
        _ranks_kernel,
        grid=(R // RB,),
        in_specs=[
            pl.BlockSpec((R, 1), lambda i: (0, 0), memory_space=pltpu.SMEM),
            pl.BlockSpec((RB, N), lambda i: (i, 0)),
            pl.BlockSpec((RB, H, 1), lambda i: (i, 0, 0)),
            pl.BlockSpec((RB, 1, H), lambda i: (i, 0, 0)),
        ],
        out_specs=pl.BlockSpec((RB, H, 1), lambda i: (i, 0, 0)),
        out_shape=jax.ShapeDtypeStruct((R, H, 1), jnp.int32),
        scratch_shapes=[pltpu.VMEM((H, H), jnp.float32)],
        compiler_params=pltpu.CompilerParams(
            dimension_semantics=("parallel",)),
    )(in_total, in_proba, hcol, hrow)

    sc_call = pl.kernel(
        _sc_scatter_body,
        out_type=jax.ShapeDtypeStruct((R, N), jnp.float32),
        mesh=plsc.VectorSubcoreMesh(core_axis_name="c", subcore_axis_name="s"),
        compiler_params=pltpu.CompilerParams(needs_layout_passes=False),
        scratch_types=[
            pltpu.VMEM((2, H), jnp.int32),
            pltpu.VMEM((H,), jnp.int32),
            pltpu.VMEM((H,), jnp.float32),
            pltpu.VMEM((N,), jnp.float32),
        ],
    )
    return sc_call(in_proba.reshape(R * N), lidx.reshape(R // 2, 2, H))


# speedup vs baseline: 1.0393x; 1.0284x over previous
import functools

import jax
import jax.numpy as jnp
from jax import lax
from jax.experimental import pallas as pl
from jax.experimental.pallas import tpu as pltpu
from jax.experimental.pallas import tpu_sc as plsc

N = 32768
H = 128
NCH = N // H


KP = 32
RB = 8


def _ranks_kernel(total_ref, x_ref, hcol_ref, hrow_ref, lidx_ref, acc_ref):
    i = pl.program_id(0)

    for r in range(RB):
        T = total_ref[i * RB + r, 0]

        for m in range(H // KP):
            @pl.when(T > m * KP)
            def _():
                hs = hcol_ref[r, pl.ds(m * KP, KP), :]
                hb = jnp.broadcast_to(hs, (KP, H))
                z = jnp.zeros((KP, H), jnp.float32)

                def body(c, accs):
                    a0, a1, a2, a3 = accs
                    x0 = x_ref[r, pl.ds((4 * c) * H, H)].reshape(1, H)
                    x1 = x_ref[r, pl.ds((4 * c + 1) * H, H)].reshape(1, H)
                    x2 = x_ref[r, pl.ds((4 * c + 2) * H, H)].reshape(1, H)
                    x3 = x_ref[r, pl.ds((4 * c + 3) * H, H)].reshape(1, H)
                    return (a0 + (x0 > hb).astype(jnp.float32),
                            a1 + (x1 > hb).astype(jnp.float32),
                            a2 + (x2 > hb).astype(jnp.float32),
                            a3 + (x3 > hb).astype(jnp.float32))

                a0, a1, a2, a3 = lax.fori_loop(0, NCH // 4, body,
                                               (z, z, z, z))
                acc_ref[pl.ds(m * KP, KP), :] = (a0 + a1) + (a2 + a3)

        hcol = hcol_ref[r]
        HB = jnp.broadcast_to(hcol, (H, H))

        hrow = hrow_ref[r]
        ks = lax.broadcasted_iota(jnp.int32, (H, H), 0)
        ml = lax.broadcasted_iota(jnp.int32, (H, H), 1)
        ties = ((HB == hrow) & (ml < ks)).astype(jnp.float32)
        total = acc_ref[...] + ties
        ranks = jnp.sum(total, axis=1, keepdims=True).astype(jnp.int32)

        kcol = lax.broadcasted_iota(jnp.int32, (H, 1), 0)
        lidx_ref[r] = jnp.where(kcol < T, ranks, -1)


def _sc_scatter_body(x_flat, lidx_hbm, out_hbm, lidx_v, gidx_v, vals_v, canvas):
    info = plsc.get_sparse_core_info()
    nc = info.num_cores
    wid = lax.axis_index("s") * nc + lax.axis_index("c")

    zeros16 = jnp.zeros((16,), jnp.float32)

    def zero_body(z, carry):
        base = z * 128
        for u in range(8):
            canvas[pl.ds(base + u * 16, 16)] = zeros16
        return carry

    lax.fori_loop(0, N // 128, zero_body, 0)

    pltpu.sync_copy(lidx_hbm.at[wid], lidx_v)

    for r in range(2):
        row = wid * 2 + r
        for s in range(H // 16):
            sl = pl.ds(s * 16, 16)
            gidx_v[sl] = jnp.maximum(lidx_v[r, sl], 0) + row * N
        pltpu.sync_copy(x_flat.at[gidx_v], vals_v)
        for s in range(H // 16):
            sl = pl.ds(s * 16, 16)
            raw = lidx_v[r, sl]
            plsc.store_scatter(canvas, [jnp.maximum(raw, 0)], vals_v[sl],
                               mask=raw >= 0)
        pltpu.sync_copy(canvas, out_hbm.at[row])
        if r == 0:
            for s in range(H // 16):
                sl = pl.ds(s * 16, 16)
                raw = lidx_v[r, sl]
                plsc.store_scatter(canvas, [jnp.maximum(raw, 0)], zeros16,
                                   mask=raw >= 0)


@jax.jit
def kernel(in_proba, in_total):
    R = in_proba.shape[0]
    hcol = in_proba[:, :H].reshape(R, H, 1)
    hrow = in_proba[:, :H].reshape(R, 1, H)

    lidx = pl.pallas_call(
        _ranks_kernel,
        grid=(R // RB,),
        in_specs=[
            pl.BlockSpec((R, 1), lambda i: (0, 0), memory_space=pltpu.SMEM),
            pl.BlockSpec((RB, N), lambda i: (i, 0)),
            pl.BlockSpec((RB, H, 1), lambda i: (i, 0, 0)),
            pl.BlockSpec((RB, 1, H), lambda i: (i, 0, 0)),
        ],
        out_specs=pl.BlockSpec((RB, H, 1), lambda i: (i, 0, 0)),
        out_shape=jax.ShapeDtypeStruct((R, H, 1), jnp.int32),
        scratch_shapes=[pltpu.VMEM((H, H), jnp.float32)],
        compiler_params=pltpu.CompilerParams(
            dimension_semantics=("parallel",)),
    )(in_total, in_proba, hcol, hrow)

    sc_call = pl.kernel(
        _sc_scatter_body,
        out_type=jax.ShapeDtypeStruct((R, N), jnp.float32),
        mesh=plsc.VectorSubcoreMesh(core_axis_name="c", subcore_axis_name="s"),
        compiler_params=pltpu.CompilerParams(needs_layout_passes=False),
        scratch_types=[
            pltpu.VMEM((2, H), jnp.int32),
            pltpu.VMEM((H,), jnp.int32),
            pltpu.VMEM((H,), jnp.float32),
            pltpu.VMEM((N,), jnp.float32),
        ],
    )
    return sc_call(in_proba.reshape(R * N), lidx.reshape(R // 2, 2, H))
